# baseline (speedup 1.0000x reference)
import functools

import jax
import jax.numpy as jnp
from jax import lax
from jax.experimental import pallas as pl
from jax.experimental.pallas import tpu as pltpu
from jax.experimental.pallas import tpu_sc as plsc

N_F = 26
V = 100000
E = 32
B = 16384

NC = 2
NS = 16
L = 16
FPC = N_F // NC
VH = 50048
VS = 3200
VS15A = 2048
VS15B = 1920
NTASK = FPC * 4
HOUT = 4096


def _body(idx_hbm, tab_hbm, tail_hbm, out_hbm, idx_v, plane, orow,
          splane, sout, gsem, wsem):
    c = lax.axis_index("c")
    s = lax.axis_index("s")
    er = lax.rem(s, 8)
    half = s // 8

    def task(i, carry):
        f = 13 * c + i // 4
        r = lax.rem(i, 4)

        def stage(vh):
            @pl.when(s < 15)
            def _():
                pltpu.sync_copy(
                    tab_hbm.at[f, pl.ds(pl.multiple_of(8 * r, 8), 8),
                               pl.ds(pl.multiple_of(vh * VH + s * VS, 128),
                                     VS)],
                    splane.at[:, pl.ds(pl.multiple_of(s * VS, 128), VS)],
                )

            if vh == 0:
                @pl.when(s == 15)
                def _():
                    pltpu.sync_copy(
                        tab_hbm.at[f, pl.ds(pl.multiple_of(8 * r, 8), 8),
                                   pl.ds(15 * VS, VS15A)],
                        splane.at[:, pl.ds(15 * VS, VS15A)],
                    )
            else:
                @pl.when(s == 15)
                def _():
                    pltpu.sync_copy(
                        tab_hbm.at[f, pl.ds(pl.multiple_of(8 * r, 8), 8),
                                   pl.ds(VH + 15 * VS, VS15B)],
                        splane.at[:, pl.ds(15 * VS, VS15B)],
                    )
                    pltpu.sync_copy(
                        tail_hbm.at[f, pl.ds(pl.multiple_of(8 * r, 8), 8), :],
                        splane.at[:, pl.ds(15 * VS + VS15B, 128)],
                    )

        def gathers(vh):
            pltpu.sync_copy(splane.at[er, :], plane)
            lo = vh * VH

            def chunk(k, carry):
                for u in range(4):
                    o = k * 4 * L + u * L
                    row = o // 128
                    col = o - row * 128
                    iv = idx_v[row, pl.ds(col, L)]
                    m = (iv >= lo) & (iv < lo + VH)
                    g = plsc.load_gather(plane, [iv - lo], mask=m)
                    if vh == 0:
                        orow[pl.ds(o, L)] = g
                    else:
                        orow[pl.ds(o, L)] = jnp.where(m, g, orow[pl.ds(o, L)])
                return carry

            lax.fori_loop(0, (B // 2) // L // 4, chunk, 0)

        pltpu.sync_copy(idx_hbm.at[f, pl.ds(half * 64, 64), :], idx_v)
        stage(0)
        plsc.subcore_barrier()
        gathers(0)
        plsc.subcore_barrier()
        stage(1)

        @pl.when((s == 0) & (i > 0))
        def _():
            for _h in range(2):
                pltpu.make_async_copy(
                    sout.at[0], out_hbm.at[0, :, pl.ds(0, B // 2)], wsem
                ).wait()

        plsc.subcore_barrier()
        gathers(1)
        pltpu.sync_copy(orow, sout.at[half, er, :])
        plsc.subcore_barrier()

        @pl.when(s == 0)
        def _():
            for h in range(2):
                pltpu.make_async_copy(
                    sout.at[h],
                    out_hbm.at[f * 4 + r, :,
                               pl.ds(pl.multiple_of(h * (B // 2), 128),
                                     B // 2)],
                    wsem,
                ).start()

        return carry

    lax.fori_loop(0, NTASK, task, 0)

    @pl.when(s == 0)
    def _():
        for _h in range(2):
            pltpu.make_async_copy(
                sout.at[0], out_hbm.at[0, :, pl.ds(0, B // 2)], wsem
            ).wait()


@functools.partial(
    pl.kernel,
    out_type=jax.ShapeDtypeStruct((N_F * E // 8, 8, B), jnp.float32),
    mesh=plsc.VectorSubcoreMesh(core_axis_name="c", subcore_axis_name="s"),
    compiler_params=pltpu.CompilerParams(
        use_tc_tiling_on_sc=True, needs_layout_passes=False
    ),
    scratch_types=[
        pltpu.VMEM((64, 128), jnp.int32),
        pltpu.VMEM((VH,), jnp.float32),
        pltpu.VMEM((B // 2,), jnp.float32),
        pltpu.VMEM_SHARED((8, VH), jnp.float32),
        pltpu.VMEM_SHARED((2, 8, B // 2), jnp.float32),
        pltpu.SemaphoreType.DMA,
        pltpu.SemaphoreType.DMA,
    ],
)
def _gather_kernel(idx_hbm, tab_hbm, tail_hbm, out_hbm, idx_v, plane, orow,
                   splane, sout, gsem, wsem):
    _body(idx_hbm, tab_hbm, tail_hbm, out_hbm, idx_v, plane, orow,
          splane, sout, gsem, wsem)


def kernel(families, tables):
    fam3 = families.astype(jnp.int32).reshape(N_F, B // 128, 128)
    tabT = jnp.transpose(tables, (0, 2, 1))
    tails = jnp.pad(
        jnp.transpose(tables[:, VH + 15 * VS + VS15B :, :], (0, 2, 1)),
        ((0, 0), (0, 0), (0, 128 - (V - VH - 15 * VS - VS15B))),
    )
    outT3 = _gather_kernel(fam3, tabT, tails)
    return outT3.reshape(N_F * E, B).T


# speedup vs baseline: 1.4657x; 1.4657x over previous
import functools

import jax
import jax.numpy as jnp
from jax import lax
from jax.experimental import pallas as pl
from jax.experimental.pallas import tpu as pltpu
from jax.experimental.pallas import tpu_sc as plsc

N_F = 26
V = 100000
E = 32
B = 16384

NC = 2
NS = 16
L = 16
HOUT = B // 4


def _body(idx_hbm, tab_hbm, out_hbm, idx_v, plane, orow, gsem, wsem):
    e = lax.axis_index("s") * NC + lax.axis_index("c")

    def p_start(f):
        pltpu.make_async_copy(tab_hbm.at[f, e, :], plane, gsem).start()

    def p_wait():
        pltpu.make_async_copy(tab_hbm.at[0, 0, :], plane, gsem).wait()

    def w_desc(f, q):
        return pltpu.make_async_copy(
            orow.at[lax.rem(q, 2)],
            out_hbm.at[f * E + e, pl.ds(q * HOUT, HOUT)],
            wsem,
        )

    p_start(0)

    def field(f, carry):
        pltpu.sync_copy(idx_hbm.at[f, :], idx_v)
        p_wait()

        def w_wait():
            pltpu.make_async_copy(
                orow.at[0], out_hbm.at[0, pl.ds(0, HOUT)], wsem
            ).wait()

        def quarter(q):
            def chunk(k, carry):
                for u in range(8):
                    o = k * 8 * L + u * L
                    iv = idx_v[pl.ds(q * HOUT + o, L)]
                    orow[q % 2, pl.ds(o, L)] = plsc.load_gather(
                        plane, [iv]
                    )
                return carry

            lax.fori_loop(0, HOUT // L // 8, chunk, 0)

            if q >= 2:
                w_wait()
            else:
                @pl.when(f > 0)
                def _():
                    w_wait()

            w_desc(f, q).start()

        for q in range(4):
            quarter(q)

        @pl.when(f + 1 < N_F)
        def _():
            p_start(f + 1)

        return carry

    lax.fori_loop(0, N_F, field, 0)
    for _ in range(2):
        pltpu.make_async_copy(
            orow.at[0], out_hbm.at[0, pl.ds(0, HOUT)], wsem
        ).wait()


@functools.partial(
    pl.kernel,
    out_type=jax.ShapeDtypeStruct((N_F * E, B), jnp.float32),
    mesh=plsc.VectorSubcoreMesh(core_axis_name="c", subcore_axis_name="s"),
    compiler_params=pltpu.CompilerParams(
        use_tc_tiling_on_sc=False, needs_layout_passes=False
    ),
    scratch_types=[
        pltpu.VMEM((B,), jnp.int32),
        pltpu.VMEM((V,), jnp.float32),
        pltpu.VMEM((2, HOUT), jnp.float32),
        pltpu.SemaphoreType.DMA,
        pltpu.SemaphoreType.DMA,
    ],
)
def _gather_kernel(idx_hbm, tab_hbm, out_hbm, idx_v, plane, orow, gsem, wsem):
    _body(idx_hbm, tab_hbm, out_hbm, idx_v, plane, orow, gsem, wsem)


def kernel(families, tables):
    fam = families.astype(jnp.int32)
    tabT = jnp.transpose(tables, (0, 2, 1))
    outT = _gather_kernel(fam, tabT)
    return outT.T
